# Initial kernel scaffold; baseline (speedup 1.0000x reference)
import jax
import jax.numpy as jnp
from jax.experimental import pallas as pl


def kernel(probs):
    raise NotImplementedError("write your pallas kernel here")


# speedup vs baseline: 20.4800x; 20.4800x over previous
import functools

import jax
import jax.numpy as jnp
from jax.experimental import pallas as pl

_KEEP_RATIO_HIGH = 0.25
_ROWS_PER_BLOCK = 8


def _topk_mask_body(k, p_ref, o_ref):
    p = p_ref[...]
    keys = jax.lax.bitcast_convert_type(p, jnp.int32)
    r, t = keys.shape
    idx = jax.lax.broadcasted_iota(jnp.int32, (r, t), 1)
    one = jnp.int32(1)

    def val_step(i, prefix):
        cand = prefix | jax.lax.shift_left(one, 30 - i)
        cnt = jnp.sum((keys >= cand).astype(jnp.int32), axis=1, keepdims=True)
        return jnp.where(cnt >= k, cand, prefix)

    thr = jax.lax.fori_loop(0, 31, val_step,
                            jnp.zeros((r, 1), jnp.int32))

    gt = keys > thr
    eq = keys == thr
    c_gt = jnp.sum(gt.astype(jnp.int32), axis=1, keepdims=True)
    need = k - c_gt

    def idx_step(i, prefix):
        cand = prefix | jax.lax.shift_left(one, 14 - i)
        cnt = jnp.sum((eq & (idx < cand)).astype(jnp.int32),
                      axis=1, keepdims=True)
        return jnp.where(cnt < need, cand, prefix)

    istar = jax.lax.fori_loop(0, 15, idx_step,
                              jnp.zeros((r, 1), jnp.int32))

    keep = gt | (eq & (idx <= istar))
    o_ref[...] = keep.astype(jnp.float32)


def kernel(probs):
    b, t = probs.shape
    k = min(max(1, int(t * _KEEP_RATIO_HIGH)), t)
    r = _ROWS_PER_BLOCK if b % _ROWS_PER_BLOCK == 0 else b
    return pl.pallas_call(
        functools.partial(_topk_mask_body, k),
        grid=(b // r,),
        in_specs=[pl.BlockSpec((r, t), lambda i: (i, 0))],
        out_specs=pl.BlockSpec((r, t), lambda i: (i, 0)),
        out_shape=jax.ShapeDtypeStruct((b, t), jnp.float32),
    )(probs)
